# baseline (speedup 1.0000x reference)
import functools

import numpy as np
import jax
import jax.numpy as jnp
from jax import lax
from jax.experimental import pallas as pl
from jax.experimental.pallas import tpu as pltpu
from jax.experimental.pallas import tpu_sc as plsc

D_MODEL = 64
MAX_POS = 128
SCALE = 8.0

NUM_CORES = 2
NUM_SUBCORES = 16
NUM_WORKERS = NUM_CORES * NUM_SUBCORES
CHUNK = 128
NBUF = 2

VOCAB = 1000000
VBLK = 128
N_FULL_BLK = VOCAB // VBLK
TAIL = VOCAB - N_FULL_BLK * VBLK
BASE_BLK = N_FULL_BLK // NUM_WORKERS
EXTRA_BLK = N_FULL_BLK % NUM_WORKERS


def _pos_encoding_np():
    position = np.arange(MAX_POS)[:, np.newaxis]
    k = np.arange(D_MODEL)[np.newaxis, :]
    i = k // 2
    angle_rates = 1 / np.power(10000, 2 * i / np.float32(D_MODEL))
    angle_rads = position * angle_rates
    angle_rads[:, 0::2] = np.sin(angle_rads[:, 0::2])
    angle_rads[:, 1::2] = np.cos(angle_rads[:, 1::2])
    return angle_rads.astype(np.float32)


_POS_T = np.ascontiguousarray(_pos_encoding_np().T)


@jax.jit
def _sc_transpose(t_t):
    mesh = plsc.VectorSubcoreMesh(core_axis_name="c", subcore_axis_name="s")

    @functools.partial(
        pl.kernel,
        mesh=mesh,
        compiler_params=pltpu.CompilerParams(
            use_tc_tiling_on_sc=True, needs_layout_passes=False),
        out_type=jax.ShapeDtypeStruct((VOCAB // 2, 128), jnp.float32),
        scratch_types=(
            [pltpu.VMEM((D_MODEL, VBLK), jnp.float32)] * NBUF
            + [pltpu.VMEM((VBLK // 2, 128), jnp.float32)] * NBUF
            + [pltpu.SemaphoreType.DMA] * (2 * NBUF)
        ),
    )
    def k(tt_hbm, tail_hbm, tp_hbm, *bufs):
        tins = bufs[:NBUF]
        touts = bufs[NBUF:2 * NBUF]
        isem = bufs[2 * NBUF:3 * NBUF]
        osem = bufs[3 * NBUF:4 * NBUF]

        wid = lax.axis_index("s") * NUM_CORES + lax.axis_index("c")
        n_my = BASE_BLK + jnp.where(wid < EXTRA_BLK, 1, 0)
        blk0 = BASE_BLK * wid + jnp.minimum(wid, EXTRA_BLK)

        def fire_in(i, b):
            v0 = pl.multiple_of((blk0 + i) * VBLK, VBLK)
            pltpu.async_copy(tt_hbm.at[:, pl.ds(v0, VBLK)], tins[b], isem[b])

        def wait_in(b):
            pltpu.make_async_copy(
                tt_hbm.at[:, pl.ds(0, VBLK)], tins[b], isem[b]).wait()

        for b in range(NBUF):
            @pl.when(b < n_my)
            def _():
                fire_in(b, b)

        def blk_body(r, carry):
            for b in range(NBUF):
                i = r * NBUF + b

                @pl.when(i < n_my)
                def _():
                    wait_in(b)

                    @pl.when(r > 0)
                    def _():
                        pltpu.make_async_copy(
                            touts[b],
                            tp_hbm.at[pl.ds(0, VBLK // 2)], osem[b]).wait()

                    def row_body(rr, c2):
                        for g in range(8):
                            sl = pl.ds(g * 16, 16)
                            ids = lax.iota(jnp.int32, 16) + (g % 4) * 16
                            col = jnp.full((16,), 2 * rr + (g // 4), jnp.int32)
                            touts[b][rr, sl] = plsc.load_gather(
                                tins[b], [ids, col])
                        return c2

                    lax.fori_loop(0, VBLK // 2, row_body, 0, unroll=2)
                    p0 = pl.multiple_of(
                        (blk0 + i) * (VBLK // 2), VBLK // 2)
                    pltpu.async_copy(
                        touts[b], tp_hbm.at[pl.ds(p0, VBLK // 2)], osem[b])

                    @pl.when(i + NBUF < n_my)
                    def _():
                        fire_in(i + NBUF, b)
            return carry

        n_rounds = (BASE_BLK + 1 + NBUF - 1) // NBUF
        lax.fori_loop(0, n_rounds, blk_body, 0, unroll=False)

        for b in range(NBUF):
            @pl.when(b < n_my)
            def _():
                pltpu.make_async_copy(
                    touts[b], tp_hbm.at[pl.ds(0, VBLK // 2)], osem[b]).wait()

        @pl.when(wid == NUM_WORKERS - 1)
        def _():
            stg = touts[0].at[pl.ds(0, TAIL // 2)]
            pltpu.sync_copy(tail_hbm, stg)
            pltpu.sync_copy(
                stg, tp_hbm.at[pl.ds(N_FULL_BLK * (VBLK // 2), TAIL // 2)])

    tail = t_t[:, VOCAB - TAIL:].T.reshape(TAIL // 2, 128)
    return k(t_t, tail)


@functools.partial(jax.jit, static_argnames=("n_seq",))
def _sc_embed(x2d, pos_t, t_rm, *, n_seq):
    seq_per_w = n_seq // NUM_WORKERS
    n_rounds = seq_per_w // NBUF

    mesh = plsc.VectorSubcoreMesh(core_axis_name="c", subcore_axis_name="s")

    @functools.partial(
        pl.kernel,
        mesh=mesh,
        compiler_params=pltpu.CompilerParams(
            use_tc_tiling_on_sc=False, needs_layout_passes=False),
        out_type=jax.ShapeDtypeStruct((n_seq, D_MODEL, MAX_POS), jnp.float32),
        scratch_types=(
            [pltpu.VMEM((seq_per_w, CHUNK), jnp.int32)]
            + [pltpu.VMEM((D_MODEL, MAX_POS), jnp.float32)]
            + [pltpu.VMEM((CHUNK, D_MODEL), jnp.float32)] * NBUF
            + [pltpu.VMEM((D_MODEL, MAX_POS), jnp.float32)] * NBUF
            + [pltpu.SemaphoreType.DMA] * (2 * NBUF)
        ),
    )
    def k(x_hbm, pos_hbm, table_hbm, out_hbm, idx_v, pos_v, *bufs):
        rows = bufs[:NBUF]
        outs = bufs[NBUF:2 * NBUF]
        gsem = bufs[2 * NBUF:3 * NBUF]
        osem = bufs[3 * NBUF:4 * NBUF]

        wid = lax.axis_index("s") * NUM_CORES + lax.axis_index("c")
        w_seq = wid * seq_per_w
        pltpu.sync_copy(pos_hbm, pos_v)
        pltpu.sync_copy(x_hbm.at[pl.ds(w_seq, seq_per_w), :], idx_v)

        def gather_start(c, b):
            pltpu.async_copy(table_hbm.at[idx_v.at[c]], rows[b], gsem[b])

        for b in range(NBUF):
            gather_start(b, b)

        def round_body(r, carry):
            for b in range(NBUF):
                c = r * NBUF + b
                pltpu.make_async_copy(
                    table_hbm.at[idx_v.at[c]], rows[b], gsem[b]).wait()

                @pl.when(r > 0)
                def _():
                    pltpu.make_async_copy(
                        outs[b], out_hbm.at[w_seq + c], osem[b]).wait()

                def col_body(f, carry2):
                    for g in range(MAX_POS // 16):
                        sl = pl.ds(g * 16, 16)
                        ids = lax.iota(jnp.int32, 16) + g * 16
                        col = jnp.full((16,), f, jnp.int32)
                        v = plsc.load_gather(rows[b], [ids, col])
                        outs[b][f, sl] = v * SCALE + pos_v[f, sl]
                    return carry2

                lax.fori_loop(0, D_MODEL, col_body, 0, unroll=2)
                pltpu.async_copy(outs[b], out_hbm.at[w_seq + c], osem[b])

                @pl.when(r < n_rounds - 1)
                def _():
                    gather_start(c + NBUF, b)
            return carry

        lax.fori_loop(0, n_rounds, round_body, 0, unroll=False)

        for b in range(NBUF):
            c = (n_rounds - 1) * NBUF + b
            pltpu.make_async_copy(
                outs[b], out_hbm.at[w_seq + c], osem[b]).wait()

    return k(x2d, pos_t, t_rm)


def kernel(x, table):
    b, s = x.shape
    pos_t = jnp.asarray(_POS_T)
    t_pairs = _sc_transpose(table.T)
    t_rm = t_pairs.reshape(VOCAB, D_MODEL)
    out = _sc_embed(x, pos_t, t_rm, n_seq=b)
    return out.transpose(0, 2, 1)


# speedup vs baseline: 1.2133x; 1.2133x over previous
import functools

import numpy as np
import jax
import jax.numpy as jnp
from jax import lax
from jax.experimental import pallas as pl
from jax.experimental.pallas import tpu as pltpu
from jax.experimental.pallas import tpu_sc as plsc

D_MODEL = 64
MAX_POS = 128
SCALE = 8.0

NUM_CORES = 2
NUM_SUBCORES = 16
NUM_WORKERS = NUM_CORES * NUM_SUBCORES
CHUNK = 128
NBUF = 2

VOCAB = 1000000
VBLK = 128
N_FULL_BLK = VOCAB // VBLK
TAIL = VOCAB - N_FULL_BLK * VBLK
BASE_BLK = N_FULL_BLK // NUM_WORKERS
EXTRA_BLK = N_FULL_BLK % NUM_WORKERS


def _pos_encoding_np():
    position = np.arange(MAX_POS)[:, np.newaxis]
    k = np.arange(D_MODEL)[np.newaxis, :]
    i = k // 2
    angle_rates = 1 / np.power(10000, 2 * i / np.float32(D_MODEL))
    angle_rads = position * angle_rates
    angle_rads[:, 0::2] = np.sin(angle_rads[:, 0::2])
    angle_rads[:, 1::2] = np.cos(angle_rads[:, 1::2])
    return angle_rads.astype(np.float32)


_POS = _pos_encoding_np()


@jax.jit
def _sc_transpose(t_t):
    mesh = plsc.VectorSubcoreMesh(core_axis_name="c", subcore_axis_name="s")

    @functools.partial(
        pl.kernel,
        mesh=mesh,
        compiler_params=pltpu.CompilerParams(
            use_tc_tiling_on_sc=True, needs_layout_passes=False),
        out_type=jax.ShapeDtypeStruct((VOCAB // 2, 128), jnp.float32),
        scratch_types=(
            [pltpu.VMEM((D_MODEL, VBLK), jnp.float32)] * NBUF
            + [pltpu.VMEM((VBLK // 2, 128), jnp.float32)] * NBUF
            + [pltpu.SemaphoreType.DMA] * (2 * NBUF)
        ),
    )
    def k(tt_hbm, tail_hbm, tp_hbm, *bufs):
        tins = bufs[:NBUF]
        touts = bufs[NBUF:2 * NBUF]
        isem = bufs[2 * NBUF:3 * NBUF]
        osem = bufs[3 * NBUF:4 * NBUF]

        wid = lax.axis_index("s") * NUM_CORES + lax.axis_index("c")
        n_my = BASE_BLK + jnp.where(wid < EXTRA_BLK, 1, 0)
        blk0 = BASE_BLK * wid + jnp.minimum(wid, EXTRA_BLK)

        def fire_in(i, b):
            v0 = pl.multiple_of((blk0 + i) * VBLK, VBLK)
            pltpu.async_copy(tt_hbm.at[:, pl.ds(v0, VBLK)], tins[b], isem[b])

        def wait_in(b):
            pltpu.make_async_copy(
                tt_hbm.at[:, pl.ds(0, VBLK)], tins[b], isem[b]).wait()

        for b in range(NBUF):
            @pl.when(b < n_my)
            def _():
                fire_in(b, b)

        def blk_body(r, carry):
            for b in range(NBUF):
                i = r * NBUF + b

                @pl.when(i < n_my)
                def _():
                    wait_in(b)

                    @pl.when(r > 0)
                    def _():
                        pltpu.make_async_copy(
                            touts[b],
                            tp_hbm.at[pl.ds(0, VBLK // 2)], osem[b]).wait()

                    lane = lax.iota(jnp.int32, 16)

                    def feat_body(f, c2):
                        cbase = jnp.full((16,), f, jnp.int32) + 64 * (lane & 1)
                        for g in range(VBLK // 16):
                            sl = pl.ds(g * 16, 16)
                            rr = (lane >> 1) + g * 8
                            v = tins[b][f, sl]
                            plsc.store_scatter(touts[b], [rr, cbase], v)
                        return c2

                    lax.fori_loop(0, D_MODEL, feat_body, 0, unroll=2)
                    p0 = pl.multiple_of(
                        (blk0 + i) * (VBLK // 2), VBLK // 2)
                    pltpu.async_copy(
                        touts[b], tp_hbm.at[pl.ds(p0, VBLK // 2)], osem[b])

                    @pl.when(i + NBUF < n_my)
                    def _():
                        fire_in(i + NBUF, b)
            return carry

        n_rounds = (BASE_BLK + 1 + NBUF - 1) // NBUF
        lax.fori_loop(0, n_rounds, blk_body, 0, unroll=False)

        for b in range(NBUF):
            @pl.when(b < n_my)
            def _():
                pltpu.make_async_copy(
                    touts[b], tp_hbm.at[pl.ds(0, VBLK // 2)], osem[b]).wait()

        @pl.when(wid == NUM_WORKERS - 1)
        def _():
            stg = touts[0].at[pl.ds(0, TAIL // 2)]
            pltpu.sync_copy(tail_hbm, stg)
            pltpu.sync_copy(
                stg, tp_hbm.at[pl.ds(N_FULL_BLK * (VBLK // 2), TAIL // 2)])

    tail = t_t[:, VOCAB - TAIL:].T.reshape(TAIL // 2, 128)
    return k(t_t, tail)


@functools.partial(jax.jit, static_argnames=("n_seq",))
def _sc_embed(x2d, pos_t, t_rm, *, n_seq):
    seq_per_w = n_seq // NUM_WORKERS
    n_rounds = seq_per_w // NBUF

    mesh = plsc.VectorSubcoreMesh(core_axis_name="c", subcore_axis_name="s")

    @functools.partial(
        pl.kernel,
        mesh=mesh,
        compiler_params=pltpu.CompilerParams(
            use_tc_tiling_on_sc=False, needs_layout_passes=False),
        out_type=jax.ShapeDtypeStruct((n_seq, D_MODEL, MAX_POS), jnp.float32),
        scratch_types=(
            [pltpu.VMEM((seq_per_w, CHUNK), jnp.int32)]
            + [pltpu.VMEM((MAX_POS, D_MODEL), jnp.float32)]
            + [pltpu.VMEM((CHUNK, D_MODEL), jnp.float32)] * NBUF
            + [pltpu.VMEM((D_MODEL, MAX_POS), jnp.float32)] * NBUF
            + [pltpu.SemaphoreType.DMA] * (2 * NBUF)
        ),
    )
    def k(x_hbm, pos_hbm, table_hbm, out_hbm, idx_v, pos_v, *bufs):
        rows = bufs[:NBUF]
        outs = bufs[NBUF:2 * NBUF]
        gsem = bufs[2 * NBUF:3 * NBUF]
        osem = bufs[3 * NBUF:4 * NBUF]

        wid = lax.axis_index("s") * NUM_CORES + lax.axis_index("c")
        w_seq = wid * seq_per_w
        pltpu.sync_copy(pos_hbm, pos_v)
        pltpu.sync_copy(x_hbm.at[pl.ds(w_seq, seq_per_w), :], idx_v)

        def gather_start(c, b):
            pltpu.async_copy(table_hbm.at[idx_v.at[c]], rows[b], gsem[b])

        for b in range(NBUF):
            gather_start(b, b)

        def round_body(r, carry):
            for b in range(NBUF):
                c = r * NBUF + b
                pltpu.make_async_copy(
                    table_hbm.at[idx_v.at[c]], rows[b], gsem[b]).wait()

                @pl.when(r > 0)
                def _():
                    pltpu.make_async_copy(
                        outs[b], out_hbm.at[w_seq + c], osem[b]).wait()

                def tok_body(t, carry2):
                    col = jnp.full((16,), t, jnp.int32)
                    for g in range(D_MODEL // 16):
                        sl = pl.ds(g * 16, 16)
                        ids = lax.iota(jnp.int32, 16) + g * 16
                        v = rows[b][t, sl] * SCALE + pos_v[t, sl]
                        plsc.store_scatter(outs[b], [ids, col], v)
                    return carry2

                lax.fori_loop(0, MAX_POS, tok_body, 0, unroll=2)
                pltpu.async_copy(outs[b], out_hbm.at[w_seq + c], osem[b])

                @pl.when(r < n_rounds - 1)
                def _():
                    gather_start(c + NBUF, b)
            return carry

        lax.fori_loop(0, n_rounds, round_body, 0, unroll=False)

        for b in range(NBUF):
            c = (n_rounds - 1) * NBUF + b
            pltpu.make_async_copy(
                outs[b], out_hbm.at[w_seq + c], osem[b]).wait()

    return k(x2d, pos_t, t_rm)


def kernel(x, table):
    b, s = x.shape
    pos_t = jnp.asarray(_POS)
    t_pairs = _sc_transpose(table.T)
    t_rm = t_pairs.reshape(VOCAB, D_MODEL)
    out = _sc_embed(x, pos_t, t_rm, n_seq=b)
    return out.transpose(0, 2, 1)


# speedup vs baseline: 1.8011x; 1.4845x over previous
import functools

import numpy as np
import jax
import jax.numpy as jnp
from jax import lax
from jax.experimental import pallas as pl
from jax.experimental.pallas import tpu as pltpu
from jax.experimental.pallas import tpu_sc as plsc

D_MODEL = 64
MAX_POS = 128
SCALE = 8.0

NUM_CORES = 2
NUM_SUBCORES = 16
NUM_WORKERS = NUM_CORES * NUM_SUBCORES
CHUNK = 128
NBUF = 2

VOCAB = 1000000
VBLK = 128
N_FULL_BLK = VOCAB // VBLK
TAIL = VOCAB - N_FULL_BLK * VBLK
BASE_BLK = N_FULL_BLK // NUM_WORKERS
EXTRA_BLK = N_FULL_BLK % NUM_WORKERS


def _pos_encoding_np():
    position = np.arange(MAX_POS)[:, np.newaxis]
    k = np.arange(D_MODEL)[np.newaxis, :]
    i = k // 2
    angle_rates = 1 / np.power(10000, 2 * i / np.float32(D_MODEL))
    angle_rads = position * angle_rates
    angle_rads[:, 0::2] = np.sin(angle_rads[:, 0::2])
    angle_rads[:, 1::2] = np.cos(angle_rads[:, 1::2])
    return angle_rads.astype(np.float32)


_POS = _pos_encoding_np()


@jax.jit
def _sc_transpose(t_t):
    mesh = plsc.VectorSubcoreMesh(core_axis_name="c", subcore_axis_name="s")

    @functools.partial(
        pl.kernel,
        mesh=mesh,
        compiler_params=pltpu.CompilerParams(
            use_tc_tiling_on_sc=True, needs_layout_passes=False),
        out_type=jax.ShapeDtypeStruct((VOCAB // 2, 128), jnp.float32),
        scratch_types=(
            [pltpu.VMEM((D_MODEL, VBLK), jnp.float32)] * NBUF
            + [pltpu.VMEM((VBLK // 2, 128), jnp.float32)] * NBUF
            + [pltpu.SemaphoreType.DMA] * (2 * NBUF)
        ),
    )
    def k(tt_hbm, tail_hbm, tp_hbm, *bufs):
        tins = bufs[:NBUF]
        touts = bufs[NBUF:2 * NBUF]
        isem = bufs[2 * NBUF:3 * NBUF]
        osem = bufs[3 * NBUF:4 * NBUF]

        wid = lax.axis_index("s") * NUM_CORES + lax.axis_index("c")
        n_my = BASE_BLK + jnp.where(wid < EXTRA_BLK, 1, 0)
        blk0 = BASE_BLK * wid + jnp.minimum(wid, EXTRA_BLK)

        def fire_in(i, b):
            v0 = pl.multiple_of((blk0 + i) * VBLK, VBLK)
            pltpu.async_copy(tt_hbm.at[:, pl.ds(v0, VBLK)], tins[b], isem[b])

        def wait_in(b):
            pltpu.make_async_copy(
                tt_hbm.at[:, pl.ds(0, VBLK)], tins[b], isem[b]).wait()

        for b in range(NBUF):
            @pl.when(b < n_my)
            def _():
                fire_in(b, b)

        def blk_body(r, carry):
            for b in range(NBUF):
                i = r * NBUF + b

                @pl.when(i < n_my)
                def _():
                    wait_in(b)

                    @pl.when(r > 0)
                    def _():
                        pltpu.make_async_copy(
                            touts[b],
                            tp_hbm.at[pl.ds(0, VBLK // 2)], osem[b]).wait()

                    lane = lax.iota(jnp.int32, 16)

                    @plsc.parallel_loop(0, D_MODEL, unroll=4)
                    def _(f):
                        cbase = jnp.full((16,), f, jnp.int32) + 64 * (lane & 1)
                        for g in range(VBLK // 16):
                            sl = pl.ds(g * 16, 16)
                            rr = (lane >> 1) + g * 8
                            v = tins[b][f, sl]
                            plsc.store_scatter(touts[b], [rr, cbase], v)
                    p0 = pl.multiple_of(
                        (blk0 + i) * (VBLK // 2), VBLK // 2)
                    pltpu.async_copy(
                        touts[b], tp_hbm.at[pl.ds(p0, VBLK // 2)], osem[b])

                    @pl.when(i + NBUF < n_my)
                    def _():
                        fire_in(i + NBUF, b)
            return carry

        n_rounds = (BASE_BLK + 1 + NBUF - 1) // NBUF
        lax.fori_loop(0, n_rounds, blk_body, 0, unroll=False)

        for b in range(NBUF):
            @pl.when(b < n_my)
            def _():
                pltpu.make_async_copy(
                    touts[b], tp_hbm.at[pl.ds(0, VBLK // 2)], osem[b]).wait()

        @pl.when(wid == NUM_WORKERS - 1)
        def _():
            stg = touts[0].at[pl.ds(0, TAIL // 2)]
            pltpu.sync_copy(tail_hbm, stg)
            pltpu.sync_copy(
                stg, tp_hbm.at[pl.ds(N_FULL_BLK * (VBLK // 2), TAIL // 2)])

    tail = t_t[:, VOCAB - TAIL:].T.reshape(TAIL // 2, 128)
    return k(t_t, tail)


@functools.partial(jax.jit, static_argnames=("n_seq",))
def _sc_embed(x2d, pos_t, t_rm, *, n_seq):
    seq_per_w = n_seq // NUM_WORKERS
    n_rounds = seq_per_w // NBUF

    mesh = plsc.VectorSubcoreMesh(core_axis_name="c", subcore_axis_name="s")

    @functools.partial(
        pl.kernel,
        mesh=mesh,
        compiler_params=pltpu.CompilerParams(
            use_tc_tiling_on_sc=False, needs_layout_passes=False),
        out_type=jax.ShapeDtypeStruct((n_seq, D_MODEL, MAX_POS), jnp.float32),
        scratch_types=(
            [pltpu.VMEM((seq_per_w, CHUNK), jnp.int32)]
            + [pltpu.VMEM((MAX_POS, D_MODEL), jnp.float32)]
            + [pltpu.VMEM((CHUNK, D_MODEL), jnp.float32)] * NBUF
            + [pltpu.VMEM((D_MODEL, MAX_POS), jnp.float32)] * NBUF
            + [pltpu.SemaphoreType.DMA] * (2 * NBUF)
        ),
    )
    def k(x_hbm, pos_hbm, table_hbm, out_hbm, idx_v, pos_v, *bufs):
        rows = bufs[:NBUF]
        outs = bufs[NBUF:2 * NBUF]
        gsem = bufs[2 * NBUF:3 * NBUF]
        osem = bufs[3 * NBUF:4 * NBUF]

        wid = lax.axis_index("s") * NUM_CORES + lax.axis_index("c")
        w_seq = wid * seq_per_w
        pltpu.sync_copy(pos_hbm, pos_v)
        pltpu.sync_copy(x_hbm.at[pl.ds(w_seq, seq_per_w), :], idx_v)

        def gather_start(c, b):
            pltpu.async_copy(table_hbm.at[idx_v.at[c]], rows[b], gsem[b])

        for b in range(NBUF):
            gather_start(b, b)

        def round_body(r, carry):
            for b in range(NBUF):
                c = r * NBUF + b
                pltpu.make_async_copy(
                    table_hbm.at[idx_v.at[c]], rows[b], gsem[b]).wait()

                @pl.when(r > 0)
                def _():
                    pltpu.make_async_copy(
                        outs[b], out_hbm.at[w_seq + c], osem[b]).wait()

                @plsc.parallel_loop(0, MAX_POS, unroll=4)
                def _(t):
                    col = jnp.full((16,), t, jnp.int32)
                    for g in range(D_MODEL // 16):
                        sl = pl.ds(g * 16, 16)
                        ids = lax.iota(jnp.int32, 16) + g * 16
                        v = rows[b][t, sl] * SCALE + pos_v[t, sl]
                        plsc.store_scatter(outs[b], [ids, col], v)
                pltpu.async_copy(outs[b], out_hbm.at[w_seq + c], osem[b])

                @pl.when(r < n_rounds - 1)
                def _():
                    gather_start(c + NBUF, b)
            return carry

        lax.fori_loop(0, n_rounds, round_body, 0, unroll=False)

        for b in range(NBUF):
            c = (n_rounds - 1) * NBUF + b
            pltpu.make_async_copy(
                outs[b], out_hbm.at[w_seq + c], osem[b]).wait()

    return k(x2d, pos_t, t_rm)


def kernel(x, table):
    b, s = x.shape
    pos_t = jnp.asarray(_POS)
    t_pairs = _sc_transpose(table.T)
    t_rm = t_pairs.reshape(VOCAB, D_MODEL)
    out = _sc_embed(x, pos_t, t_rm, n_seq=b)
    return out.transpose(0, 2, 1)
